# baseline (speedup 1.0000x reference)
import functools

import jax
import jax.numpy as jnp
from jax import lax
from jax.experimental import pallas as pl
from jax.experimental.pallas import tpu as pltpu
from jax.experimental.pallas import tpu_sc as plsc

N = 10000
E = 320000
F = 128

NC = 2
NS = 16
NW = NC * NS
CHUNK = 128
T_CH = 80
E_PAD = NW * CHUNK * T_CH
RPT = 624
RPT_LAST = N - (NS - 1) * RPT
N_ACC = N + 8
DW = F

_mesh = plsc.VectorSubcoreMesh(core_axis_name="c", subcore_axis_name="s")


def _stripe(s, fn):
    @pl.when(s < NS - 1)
    def _():
        fn(pl.multiple_of(s * RPT, 8), RPT)

    @pl.when(s == NS - 1)
    def _():
        fn((NS - 1) * RPT, RPT_LAST)


def _deg_body(dst_hbm, zeros_hbm, ones_hbm, out_hbm, di2d, ones_v, acc):
    c = lax.axis_index("c")
    s = lax.axis_index("s")
    wid = c * NS + s
    _stripe(s, lambda r0, nr: pltpu.sync_copy(
        zeros_hbm.at[pl.ds(r0, nr)], acc.at[pl.ds(r0, nr)]))
    pltpu.sync_copy(ones_hbm, ones_v)
    pltpu.sync_copy(dst_hbm.at[wid], di2d)
    plsc.subcore_barrier()

    def step(i, carry):
        pltpu.sync_copy(ones_v, acc.at[di2d.at[i]], add=True)
        return carry

    lax.fori_loop(0, T_CH, step, 0)
    plsc.subcore_barrier()
    _stripe(s, lambda r0, nr: pltpu.sync_copy(
        acc.at[pl.ds(r0, nr)], out_hbm.at[c, pl.ds(r0, nr)]))


_deg_call = functools.partial(
    pl.kernel,
    out_type=jax.ShapeDtypeStruct((NC, N, DW), jnp.float32),
    mesh=_mesh,
    scratch_types=[
        pltpu.VMEM((T_CH, CHUNK), jnp.int32),
        pltpu.VMEM((CHUNK, DW), jnp.float32),
        pltpu.VMEM_SHARED((N_ACC, DW), jnp.float32),
    ],
)(_deg_body)


def _edge_body(src_hbm, dst_hbm, g_hbm, zeros_hbm, out_hbm,
               si0, si1, di0, di1, rows0, rows1, acc,
               iss0, iss1, ids0, ids1, gs0, gs1):
    c = lax.axis_index("c")
    s = lax.axis_index("s")
    wid = c * NS + s

    @pl.when(c == 0)
    def _():
        _stripe(s, lambda r0, nr: pltpu.sync_copy(
            g_hbm.at[pl.ds(r0, nr)], acc.at[pl.ds(r0, nr)]))

    @pl.when(c == 1)
    def _():
        _stripe(s, lambda r0, nr: pltpu.sync_copy(
            zeros_hbm.at[pl.ds(r0, nr)], acc.at[pl.ds(r0, nr)]))

    plsc.subcore_barrier()

    si = (si0, si1)
    di = (di0, di1)
    rows = (rows0, rows1)
    iss = (iss0, iss1)
    ids = (ids0, ids1)
    gs = (gs0, gs1)
    base = wid * (T_CH * CHUNK)

    def off(i):
        return pl.multiple_of(base + i * CHUNK, CHUNK)

    pltpu.sync_copy(src_hbm.at[pl.ds(off(0), CHUNK)], si0)
    pltpu.async_copy(g_hbm.at[si0], rows0, gs0)
    pltpu.async_copy(src_hbm.at[pl.ds(off(1), CHUNK)], si1, iss1)
    pltpu.async_copy(dst_hbm.at[pl.ds(off(0), CHUNK)], di0, ids0)
    pltpu.async_copy(dst_hbm.at[pl.ds(off(1), CHUNK)], di1, ids1)

    def outer(g, carry):
        for b in range(2):
            i = g * 2 + b
            nb = 1 - b
            pltpu.make_async_copy(g_hbm.at[si[b]], rows[b], gs[b]).wait()

            @pl.when(i + 1 < T_CH)
            def _(i=i, b=b, nb=nb):
                pltpu.make_async_copy(
                    src_hbm.at[pl.ds(off(i + 1), CHUNK)], si[nb], iss[nb]).wait()
                pltpu.async_copy(g_hbm.at[si[nb]], rows[nb], gs[nb])

            pltpu.make_async_copy(
                dst_hbm.at[pl.ds(off(i), CHUNK)], di[b], ids[b]).wait()
            pltpu.sync_copy(rows[b], acc.at[di[b]], add=True)

            @pl.when(i + 2 < T_CH)
            def _(i=i, b=b):
                pltpu.async_copy(
                    src_hbm.at[pl.ds(off(i + 2), CHUNK)], si[b], iss[b])
                pltpu.async_copy(
                    dst_hbm.at[pl.ds(off(i + 2), CHUNK)], di[b], ids[b])
        return carry

    lax.fori_loop(0, T_CH // 2, outer, 0)
    plsc.subcore_barrier()
    _stripe(s, lambda r0, nr: pltpu.sync_copy(
        acc.at[pl.ds(r0, nr)], out_hbm.at[c, pl.ds(r0, nr)]))


_edge_call = functools.partial(
    pl.kernel,
    out_type=jax.ShapeDtypeStruct((NC, N, F), jnp.float32),
    mesh=_mesh,
    scratch_types=[
        pltpu.VMEM((CHUNK,), jnp.int32),
        pltpu.VMEM((CHUNK,), jnp.int32),
        pltpu.VMEM((CHUNK,), jnp.int32),
        pltpu.VMEM((CHUNK,), jnp.int32),
        pltpu.VMEM((CHUNK, F), jnp.float32),
        pltpu.VMEM((CHUNK, F), jnp.float32),
        pltpu.VMEM_SHARED((N_ACC, F), jnp.float32),
        pltpu.SemaphoreType.DMA,
        pltpu.SemaphoreType.DMA,
        pltpu.SemaphoreType.DMA,
        pltpu.SemaphoreType.DMA,
        pltpu.SemaphoreType.DMA,
        pltpu.SemaphoreType.DMA,
    ],
)(_edge_body)


BR = 1000


def _dinv_of(cnt_ref):
    return lax.rsqrt(cnt_ref[0][:, 0:1] + 1.0)


def _tc_first_body(x_ref, w_ref, c0_ref, c1_ref, o_ref):
    dinv = lax.rsqrt(c0_ref[0][:, 0:1] + c1_ref[0][:, 0:1] + 1.0)
    h = jnp.dot(x_ref[...], w_ref[...], preferred_element_type=jnp.float32)
    o_ref[...] = h * dinv


def _tc_mid_body(s0_ref, s1_ref, w_ref, b_ref, c0_ref, c1_ref, o_ref):
    dinv = lax.rsqrt(c0_ref[0][:, 0:1] + c1_ref[0][:, 0:1] + 1.0)
    a = jnp.tanh((s0_ref[0] + s1_ref[0]) * dinv + b_ref[...])
    h = jnp.dot(a, w_ref[...], preferred_element_type=jnp.float32)
    o_ref[...] = h * dinv


def _tc_last_body(s0_ref, s1_ref, b_ref, c0_ref, c1_ref, o_ref):
    dinv = lax.rsqrt(c0_ref[0][:, 0:1] + c1_ref[0][:, 0:1] + 1.0)
    o_ref[...] = jnp.tanh((s0_ref[0] + s1_ref[0]) * dinv + b_ref[...])


def _spec_rows():
    return pl.BlockSpec((BR, F), lambda i: (i, 0))


def _spec_plane(p):
    return pl.BlockSpec((1, BR, F), lambda i, p=p: (p, i, 0))


def _spec_cnt(p):
    return pl.BlockSpec((1, BR, DW), lambda i, p=p: (p, i, 0))


def _spec_w():
    return pl.BlockSpec((F, F), lambda i: (0, 0))


def _spec_b():
    return pl.BlockSpec((1, F), lambda i: (0, 0))


_out_nf = jax.ShapeDtypeStruct((N, F), jnp.float32)
_grid = (N // BR,)

_tc_first = pl.pallas_call(
    _tc_first_body, grid=_grid,
    in_specs=[_spec_rows(), _spec_w(), _spec_cnt(0), _spec_cnt(1)],
    out_specs=_spec_rows(), out_shape=_out_nf)

_tc_mid = pl.pallas_call(
    _tc_mid_body, grid=_grid,
    in_specs=[_spec_plane(0), _spec_plane(1), _spec_w(), _spec_b(),
              _spec_cnt(0), _spec_cnt(1)],
    out_specs=_spec_rows(), out_shape=_out_nf)

_tc_last = pl.pallas_call(
    _tc_last_body, grid=_grid,
    in_specs=[_spec_plane(0), _spec_plane(1), _spec_b(),
              _spec_cnt(0), _spec_cnt(1)],
    out_specs=_spec_rows(), out_shape=_out_nf)


@jax.jit
def _run(x, src, dst, W0, b0, W1, b1, W2, b2, W3, b3):
    pad = E_PAD - E
    src_p = jnp.concatenate([src, jnp.zeros((pad,), jnp.int32)])
    dst_p = jnp.concatenate([dst, jnp.full((pad,), N, jnp.int32)])
    zeros_nf = jnp.zeros((N, F), jnp.float32)
    ones_chunk = jnp.ones((CHUNK, DW), jnp.float32)

    cnt = _deg_call(dst_p.reshape(NW, T_CH, CHUNK), zeros_nf, ones_chunk)

    g = _tc_first(x, W0, cnt, cnt)
    s = _edge_call(src_p, dst_p, g, zeros_nf)
    g = _tc_mid(s, s, W1, b0.reshape(1, F), cnt, cnt)
    s = _edge_call(src_p, dst_p, g, zeros_nf)
    g = _tc_mid(s, s, W2, b1.reshape(1, F), cnt, cnt)
    s = _edge_call(src_p, dst_p, g, zeros_nf)
    g = _tc_mid(s, s, W3, b2.reshape(1, F), cnt, cnt)
    s = _edge_call(src_p, dst_p, g, zeros_nf)
    return _tc_last(s, s, b3.reshape(1, F), cnt, cnt)


def kernel(x, edge_index, W0, b0, W1, b1, W2, b2, W3, b3):
    src = edge_index[0].astype(jnp.int32)
    dst = edge_index[1].astype(jnp.int32)
    return _run(x, src, dst, W0, b0, W1, b1, W2, b2, W3, b3)


# speedup vs baseline: 1.0015x; 1.0015x over previous
import functools

import jax
import jax.numpy as jnp
from jax import lax
from jax.experimental import pallas as pl
from jax.experimental.pallas import tpu as pltpu
from jax.experimental.pallas import tpu_sc as plsc

N = 10000
E = 320000
F = 128

NC = 2
NS = 16
NW = NC * NS
CHUNK = 128
T_CH = 80
E_PAD = NW * CHUNK * T_CH
RPT = 624
RPT_LAST = N - (NS - 1) * RPT
N_ACC = N + 8
DW = F

_mesh = plsc.VectorSubcoreMesh(core_axis_name="c", subcore_axis_name="s")


def _stripe(s, fn):
    @pl.when(s < NS - 1)
    def _():
        fn(pl.multiple_of(s * RPT, 8), RPT)

    @pl.when(s == NS - 1)
    def _():
        fn((NS - 1) * RPT, RPT_LAST)


def _deg_body(dst_hbm, zeros_hbm, ones_hbm, out_hbm, di2d, ones_v, acc):
    c = lax.axis_index("c")
    s = lax.axis_index("s")
    wid = c * NS + s
    _stripe(s, lambda r0, nr: pltpu.sync_copy(
        zeros_hbm.at[pl.ds(r0, nr)], acc.at[pl.ds(r0, nr)]))
    pltpu.sync_copy(ones_hbm, ones_v)
    pltpu.sync_copy(dst_hbm.at[wid], di2d)
    plsc.subcore_barrier()

    def step(i, carry):
        pltpu.sync_copy(ones_v, acc.at[di2d.at[i]], add=True)
        return carry

    lax.fori_loop(0, T_CH, step, 0)
    plsc.subcore_barrier()
    _stripe(s, lambda r0, nr: pltpu.sync_copy(
        acc.at[pl.ds(r0, nr)], out_hbm.at[c, pl.ds(r0, nr)]))


_deg_call = functools.partial(
    pl.kernel,
    out_type=jax.ShapeDtypeStruct((NC, N, DW), jnp.float32),
    mesh=_mesh,
    scratch_types=[
        pltpu.VMEM((T_CH, CHUNK), jnp.int32),
        pltpu.VMEM((CHUNK, DW), jnp.float32),
        pltpu.VMEM_SHARED((N_ACC, DW), jnp.float32),
    ],
)(_deg_body)


BLK = 8
NBLK = T_CH // BLK


def _edge_body(src_hbm, dst_hbm, g_hbm, zeros_hbm, out_hbm,
               sb0, sb1, db0, db1, rows0, rows1, acc,
               sbs0, sbs1, dbs0, dbs1, gs0, gs1):
    c = lax.axis_index("c")
    s = lax.axis_index("s")
    wid = c * NS + s

    @pl.when(c == 0)
    def _():
        _stripe(s, lambda r0, nr: pltpu.sync_copy(
            g_hbm.at[pl.ds(r0, nr)], acc.at[pl.ds(r0, nr)]))

    @pl.when(c == 1)
    def _():
        _stripe(s, lambda r0, nr: pltpu.sync_copy(
            zeros_hbm.at[pl.ds(r0, nr)], acc.at[pl.ds(r0, nr)]))

    plsc.subcore_barrier()

    sb = (sb0, sb1)
    db = (db0, db1)
    sbs = (sbs0, sbs1)
    dbs = (dbs0, dbs1)
    rows = (rows0, rows1)
    gs = (gs0, gs1)

    def blk_off(j):
        return pl.multiple_of(j * BLK, BLK)

    def fire_blk(j, p):
        pltpu.async_copy(src_hbm.at[wid, pl.ds(blk_off(j), BLK)], sb[p], sbs[p])
        pltpu.async_copy(dst_hbm.at[wid, pl.ds(blk_off(j), BLK)], db[p], dbs[p])

    def wait_sblk(j, p):
        pltpu.make_async_copy(
            src_hbm.at[wid, pl.ds(blk_off(j), BLK)], sb[p], sbs[p]).wait()

    def wait_dblk(j, p):
        pltpu.make_async_copy(
            dst_hbm.at[wid, pl.ds(blk_off(j), BLK)], db[p], dbs[p]).wait()

    fire_blk(0, 0)
    fire_blk(1, 1)
    wait_sblk(0, 0)
    wait_dblk(0, 0)
    pltpu.async_copy(g_hbm.at[sb0.at[0]], rows0, gs0)

    def outer(j2, carry):
        for jj in range(2):
            j = j2 * 2 + jj
            for u in range(BLK):
                b = u % 2
                nb = 1 - b
                pltpu.make_async_copy(
                    g_hbm.at[sb[jj].at[u]], rows[b], gs[b]).wait()
                if u < BLK - 1:
                    pltpu.async_copy(
                        g_hbm.at[sb[jj].at[u + 1]], rows[nb], gs[nb])
                else:
                    @pl.when(j + 1 < NBLK)
                    def _(j=j, jj=jj, nb=nb):
                        wait_sblk(j + 1, 1 - jj)
                        pltpu.async_copy(
                            g_hbm.at[sb[1 - jj].at[0]], rows[nb], gs[nb])
                pltpu.sync_copy(rows[b], acc.at[db[jj].at[u]], add=True)
            @pl.when(j + 2 < NBLK)
            def _(j=j, jj=jj):
                fire_blk(j + 2, jj)

            @pl.when(j + 1 < NBLK)
            def _(j=j, jj=jj):
                wait_dblk(j + 1, 1 - jj)
        return carry

    lax.fori_loop(0, NBLK // 2, outer, 0)
    plsc.subcore_barrier()
    _stripe(s, lambda r0, nr: pltpu.sync_copy(
        acc.at[pl.ds(r0, nr)], out_hbm.at[c, pl.ds(r0, nr)]))


_edge_call = functools.partial(
    pl.kernel,
    out_type=jax.ShapeDtypeStruct((NC, N, F), jnp.float32),
    mesh=_mesh,
    scratch_types=[
        pltpu.VMEM((BLK, CHUNK), jnp.int32),
        pltpu.VMEM((BLK, CHUNK), jnp.int32),
        pltpu.VMEM((BLK, CHUNK), jnp.int32),
        pltpu.VMEM((BLK, CHUNK), jnp.int32),
        pltpu.VMEM((CHUNK, F), jnp.float32),
        pltpu.VMEM((CHUNK, F), jnp.float32),
        pltpu.VMEM_SHARED((N_ACC, F), jnp.float32),
        pltpu.SemaphoreType.DMA,
        pltpu.SemaphoreType.DMA,
        pltpu.SemaphoreType.DMA,
        pltpu.SemaphoreType.DMA,
        pltpu.SemaphoreType.DMA,
        pltpu.SemaphoreType.DMA,
    ],
)(_edge_body)


BR = 1000


def _dinv_of(cnt_ref):
    return lax.rsqrt(cnt_ref[0][:, 0:1] + 1.0)


def _tc_first_body(x_ref, w_ref, c0_ref, c1_ref, o_ref):
    dinv = lax.rsqrt(c0_ref[0][:, 0:1] + c1_ref[0][:, 0:1] + 1.0)
    h = jnp.dot(x_ref[...], w_ref[...], preferred_element_type=jnp.float32)
    o_ref[...] = h * dinv


def _tc_mid_body(s0_ref, s1_ref, w_ref, b_ref, c0_ref, c1_ref, o_ref):
    dinv = lax.rsqrt(c0_ref[0][:, 0:1] + c1_ref[0][:, 0:1] + 1.0)
    a = jnp.tanh((s0_ref[0] + s1_ref[0]) * dinv + b_ref[...])
    h = jnp.dot(a, w_ref[...], preferred_element_type=jnp.float32)
    o_ref[...] = h * dinv


def _tc_last_body(s0_ref, s1_ref, b_ref, c0_ref, c1_ref, o_ref):
    dinv = lax.rsqrt(c0_ref[0][:, 0:1] + c1_ref[0][:, 0:1] + 1.0)
    o_ref[...] = jnp.tanh((s0_ref[0] + s1_ref[0]) * dinv + b_ref[...])


def _spec_rows():
    return pl.BlockSpec((BR, F), lambda i: (i, 0))


def _spec_plane(p):
    return pl.BlockSpec((1, BR, F), lambda i, p=p: (p, i, 0))


def _spec_cnt(p):
    return pl.BlockSpec((1, BR, DW), lambda i, p=p: (p, i, 0))


def _spec_w():
    return pl.BlockSpec((F, F), lambda i: (0, 0))


def _spec_b():
    return pl.BlockSpec((1, F), lambda i: (0, 0))


_out_nf = jax.ShapeDtypeStruct((N, F), jnp.float32)
_grid = (N // BR,)

_tc_first = pl.pallas_call(
    _tc_first_body, grid=_grid,
    in_specs=[_spec_rows(), _spec_w(), _spec_cnt(0), _spec_cnt(1)],
    out_specs=_spec_rows(), out_shape=_out_nf)

_tc_mid = pl.pallas_call(
    _tc_mid_body, grid=_grid,
    in_specs=[_spec_plane(0), _spec_plane(1), _spec_w(), _spec_b(),
              _spec_cnt(0), _spec_cnt(1)],
    out_specs=_spec_rows(), out_shape=_out_nf)

_tc_last = pl.pallas_call(
    _tc_last_body, grid=_grid,
    in_specs=[_spec_plane(0), _spec_plane(1), _spec_b(),
              _spec_cnt(0), _spec_cnt(1)],
    out_specs=_spec_rows(), out_shape=_out_nf)


@jax.jit
def _run(x, src, dst, W0, b0, W1, b1, W2, b2, W3, b3):
    pad = E_PAD - E
    src_p = jnp.concatenate(
        [src, jnp.zeros((pad,), jnp.int32)]).reshape(NW, T_CH, CHUNK)
    dst_p = jnp.concatenate(
        [dst, jnp.full((pad,), N, jnp.int32)]).reshape(NW, T_CH, CHUNK)
    zeros_nf = jnp.zeros((N, F), jnp.float32)
    ones_chunk = jnp.ones((CHUNK, DW), jnp.float32)

    cnt = _deg_call(dst_p, zeros_nf, ones_chunk)

    g = _tc_first(x, W0, cnt, cnt)
    s = _edge_call(src_p, dst_p, g, zeros_nf)
    g = _tc_mid(s, s, W1, b0.reshape(1, F), cnt, cnt)
    s = _edge_call(src_p, dst_p, g, zeros_nf)
    g = _tc_mid(s, s, W2, b1.reshape(1, F), cnt, cnt)
    s = _edge_call(src_p, dst_p, g, zeros_nf)
    g = _tc_mid(s, s, W3, b2.reshape(1, F), cnt, cnt)
    s = _edge_call(src_p, dst_p, g, zeros_nf)
    return _tc_last(s, s, b3.reshape(1, F), cnt, cnt)


def kernel(x, edge_index, W0, b0, W1, b1, W2, b2, W3, b3):
    src = edge_index[0].astype(jnp.int32)
    dst = edge_index[1].astype(jnp.int32)
    return _run(x, src, dst, W0, b0, W1, b1, W2, b2, W3, b3)
